# baseline (speedup 1.0000x reference)
import functools

import jax
import jax.numpy as jnp
import numpy as np
from jax import lax
from jax.experimental import pallas as pl
from jax.experimental.pallas import tpu as pltpu
from jax.experimental.pallas import tpu_sc as plsc

KNN = 32
DELTA = 1e-3
GSZ = 128
NCAND = 40
KBLK = 2048
BIGF = np.float32(3.0e38)
PADF = np.float32(1.0e30)
BIGI = np.int32(2147483647)


def _k1_body(nkeys, q_ref, k_ref, e_ref, m_ref):
    i = pl.program_id(0)
    q = q_ref[...]
    k = k_ref[...]
    Q = q.shape[0]
    gpb = KBLK // GSZ
    ones = jnp.ones((8, q.shape[1]), jnp.float32)

    acc = lax.dot_general(q, k, (((1,), (1,)), ((), ())),
                          preferred_element_type=jnp.float32)
    ksq_l = lax.dot_general(ones, k * k, (((1,), (1,)), ((), ())),
                            preferred_element_type=jnp.float32,
                            precision=lax.Precision.HIGHEST)[0:1]
    col = i * KBLK + lax.broadcasted_iota(jnp.int32, (1, KBLK), 1)
    e = jnp.where(col >= nkeys, PADF, ksq_l - 2.0 * acc)
    for g in range(gpb):
        e_ref[:, g] = e[:, g * GSZ:(g + 1) * GSZ].reshape(Q // 8, 8, GSZ)

    acc_t = lax.dot_general(k, q, (((1,), (1,)), ((), ())),
                            preferred_element_type=jnp.float32)
    ksq_s = jnp.sum(k * k, axis=1, keepdims=True)
    row = i * KBLK + lax.broadcasted_iota(jnp.int32, (KBLK, 1), 0)
    e_t = jnp.where(row >= nkeys, PADF, ksq_s - 2.0 * acc_t)
    m_ref[...] = jnp.min(e_t.reshape(gpb, GSZ, Q), axis=1)


def _k1_call(q, keys, nkeys):
    Q, D = q.shape
    nblk = (nkeys + KBLK - 1) // KBLK
    npad = nblk * KBLK
    gpb = KBLK // GSZ
    ng = npad // GSZ
    return pl.pallas_call(
        functools.partial(_k1_body, nkeys),
        grid=(nblk,),
        in_specs=[
            pl.BlockSpec((Q, D), lambda i: (0, 0)),
            pl.BlockSpec((KBLK, D), lambda i: (i, 0)),
        ],
        out_specs=[
            pl.BlockSpec((Q // 8, gpb, 8, GSZ), lambda i: (0, i, 0, 0)),
            pl.BlockSpec((gpb, Q), lambda i: (i, 0)),
        ],
        out_shape=[
            jax.ShapeDtypeStruct((Q // 8, ng, 8, GSZ), jnp.float32),
            jax.ShapeDtypeStruct((ng, Q), jnp.float32),
        ],
    )(q, keys)


def _k2_body(m_ref, rowt_ref, sm_ref, lv_ref, ll_ref):
    ng, Q = m_ref.shape
    ns = 8
    sl = ng // ns
    sm0 = jnp.concatenate(
        [jnp.min(m_ref[c * sl:(c + 1) * sl, :], axis=0, keepdims=True)
         for c in range(ns)], axis=0)
    sm_ref[...] = sm0
    lv_ref[...] = jnp.full((ns, Q), -BIGF, jnp.float32)
    ll_ref[...] = jnp.full((ns, Q), -1, jnp.int32)
    rowt_ref[...] = jnp.zeros((NCAND, Q), jnp.int32)

    def body(j, _):
        sm = sm_ref[...]
        ios = lax.broadcasted_iota(jnp.int32, (ns, Q), 0)
        v = jnp.min(sm, axis=0)
        js = jnp.min(jnp.where(sm == v[None, :], ios, BIGI), axis=0)
        oh = ios == js[None, :]
        cw = jnp.zeros((sl, Q), jnp.float32)
        for c in range(ns):
            cw = cw + jnp.where(oh[c:c + 1, :],
                                m_ref[c * sl:(c + 1) * sl, :],
                                jnp.float32(0.0))
        lvj = jnp.max(jnp.where(oh, lv_ref[...], -BIGF), axis=0)
        llj = jnp.max(jnp.where(oh, ll_ref[...], -1), axis=0)
        iol = lax.broadcasted_iota(jnp.int32, (sl, Q), 0)
        rem = (cw > lvj[None, :]) | ((cw == lvj[None, :]) & (iol > llj[None, :]))
        pin = jnp.min(jnp.where((cw == v[None, :]) & rem, iol, BIGI), axis=0)
        p = js * sl + pin
        rem2 = (cw > v[None, :]) | ((cw == v[None, :]) & (iol > pin[None, :]))
        nm = jnp.min(jnp.where(rem2, cw, BIGF), axis=0)
        sm_ref[...] = jnp.where(oh, nm[None, :], sm)
        lv_ref[...] = jnp.where(oh, v[None, :], lv_ref[...])
        ll_ref[...] = jnp.where(oh, pin[None, :], ll_ref[...])
        ioj = lax.broadcasted_iota(jnp.int32, (NCAND, Q), 0)
        qio = lax.broadcasted_iota(jnp.int32, (NCAND, Q), 1)
        r = ((qio >> 3) * ng + p[None, :]) * 8 + (qio & 7)
        rowt_ref[...] = jnp.where(ioj == j, r, rowt_ref[...])
        return 0

    lax.fori_loop(0, NCAND, body, 0)


def _k2_call(mt):
    ng, Q = mt.shape
    ns = 8
    return pl.pallas_call(
        _k2_body,
        out_shape=jax.ShapeDtypeStruct((NCAND, Q), jnp.int32),
        scratch_shapes=[
            pltpu.VMEM((ns, Q), jnp.float32),
            pltpu.VMEM((ns, Q), jnp.float32),
            pltpu.VMEM((ns, Q), jnp.int32),
        ],
    )(mt)


def _k3_gather(d2rows, rowidx_flat):
    nrows_out = rowidx_flat.shape[0]
    info = plsc.get_sparse_core_info()
    nw = info.num_cores * info.num_subcores
    b_per_w = nrows_out // nw
    chunk = 128
    nhalf = 2
    half = b_per_w // nhalf
    nchunk = half // chunk
    mesh = plsc.VectorSubcoreMesh(core_axis_name="c", subcore_axis_name="s")

    @functools.partial(
        pl.kernel, mesh=mesh,
        out_type=jax.ShapeDtypeStruct((nrows_out, GSZ), jnp.float32),
        compiler_params=pltpu.CompilerParams(use_tc_tiling_on_sc=False),
        scratch_types=[
            pltpu.VMEM((b_per_w,), jnp.int32),
            pltpu.VMEM((half, GSZ), jnp.float32),
            pltpu.SemaphoreType.DMA,
        ],
    )
    def k(tab_hbm, idx_hbm, out_hbm, idx_v, rows_v, sem):
        wid = lax.axis_index("s") * info.num_cores + lax.axis_index("c")
        pltpu.sync_copy(idx_hbm.at[pl.ds(wid * b_per_w, b_per_w)], idx_v)
        for h in range(nhalf):
            copies = []
            for c in range(nchunk):
                copies.append(pltpu.async_copy(
                    tab_hbm.at[idx_v.at[pl.ds(h * half + c * chunk, chunk)]],
                    rows_v.at[pl.ds(c * chunk, chunk)], sem))
            for cp in copies:
                cp.wait()
            pltpu.sync_copy(
                rows_v, out_hbm.at[pl.ds(wid * b_per_w + h * half, half)])

    return k(d2rows, rowidx_flat)


def _k4_body(ng, cand_ref, rowidx_ref, q_ref, d_ref, r_ref, l_ref,
             sm_ref, lv_ref, ll_ref, p_ref):
    Q, S, G = cand_ref.shape
    qbase = pl.program_id(0) * Q
    sm_ref[...] = jnp.concatenate(
        [jnp.min(cand_ref[:, s, :], axis=1, keepdims=True) for s in range(S)],
        axis=1)
    lv_ref[...] = jnp.full((Q, S), -BIGF, jnp.float32)
    ll_ref[...] = jnp.full((Q, S), -1, jnp.int32)
    d_ref[...] = jnp.zeros((Q, KNN), jnp.float32)
    p_ref[...] = jnp.zeros((Q, KNN), jnp.int32)

    def body(j, _):
        sm = sm_ref[...]
        ios = lax.broadcasted_iota(jnp.int32, (Q, S), 1)
        v = jnp.min(sm, axis=1)
        js = jnp.min(jnp.where(sm == v[:, None], ios, BIGI), axis=1)
        oh = ios == js[:, None]
        cw = jnp.zeros((Q, G), jnp.float32)
        for s in range(S):
            cw = cw + jnp.where(oh[:, s:s + 1], cand_ref[:, s, :],
                                jnp.float32(0.0))
        lvj = jnp.max(jnp.where(oh, lv_ref[...], -BIGF), axis=1)
        llj = jnp.max(jnp.where(oh, ll_ref[...], -1), axis=1)
        iol = lax.broadcasted_iota(jnp.int32, (Q, G), 1)
        rem = (cw > lvj[:, None]) | ((cw == lvj[:, None]) & (iol > llj[:, None]))
        pin = jnp.min(jnp.where((cw == v[:, None]) & rem, iol, BIGI), axis=1)
        rem2 = (cw > v[:, None]) | ((cw == v[:, None]) & (iol > pin[:, None]))
        nm = jnp.min(jnp.where(rem2, cw, BIGF), axis=1)
        sm_ref[...] = jnp.where(oh, nm[:, None], sm)
        lv_ref[...] = jnp.where(oh, v[:, None], lv_ref[...])
        ll_ref[...] = jnp.where(oh, pin[:, None], ll_ref[...])
        p = js * G + pin
        ioj = lax.broadcasted_iota(jnp.int32, (Q, KNN), 1)
        d_ref[...] = jnp.where(ioj == j, v[:, None], d_ref[...])
        p_ref[...] = jnp.where(ioj == j, p[:, None], p_ref[...])
        return 0

    lax.fori_loop(0, KNN, body, 0)

    qv = q_ref[...]
    qsq = jnp.sum(qv * qv, axis=1, keepdims=True)
    qio = qbase + lax.broadcasted_iota(jnp.int32, (Q, NCAND), 0)
    g_tab = (rowidx_ref[...] - (qio & 7)) // 8 - (qio >> 3) * ng
    pacc = p_ref[...]
    j2 = pacc // G
    off = pacc % G
    gsel = jnp.zeros((Q, KNN), jnp.int32)
    for s in range(NCAND):
        gsel = jnp.where(j2 == s, g_tab[:, s:s + 1], gsel)
    gkey = gsel * GSZ + off
    d_ref[...] = d_ref[...] + qsq
    r_ref[...] = gkey >> 4
    l_ref[...] = gkey & 15


def _k4_call(cand3, rowidx, q, ng):
    Q, S, G = cand3.shape
    D = q.shape[1]
    nqb = 1
    QB = Q // nqb
    return pl.pallas_call(
        functools.partial(_k4_body, ng),
        grid=(nqb,),
        in_specs=[
            pl.BlockSpec((QB, S, G), lambda i: (i, 0, 0)),
            pl.BlockSpec((QB, S), lambda i: (i, 0)),
            pl.BlockSpec((QB, D), lambda i: (i, 0)),
        ],
        out_specs=[
            pl.BlockSpec((QB, KNN), lambda i: (i, 0)),
            pl.BlockSpec((QB, KNN), lambda i: (i, 0)),
            pl.BlockSpec((QB, KNN), lambda i: (i, 0)),
        ],
        out_shape=[
            jax.ShapeDtypeStruct((Q, KNN), jnp.float32),
            jax.ShapeDtypeStruct((Q, KNN), jnp.int32),
            jax.ShapeDtypeStruct((Q, KNN), jnp.int32),
        ],
        scratch_shapes=[
            pltpu.VMEM((QB, S), jnp.float32),
            pltpu.VMEM((QB, S), jnp.float32),
            pltpu.VMEM((QB, S), jnp.int32),
            pltpu.VMEM((QB, KNN), jnp.int32),
        ],
    )(cand3, rowidx, q)


def _k5_gather(v16, rsel_flat):
    nb = rsel_flat.shape[0]
    L = v16.shape[1]
    info = plsc.get_sparse_core_info()
    nw = info.num_cores * info.num_subcores
    b_per_w = nb // nw
    chunk = 128
    nchunk = b_per_w // chunk
    mesh = plsc.VectorSubcoreMesh(core_axis_name="c", subcore_axis_name="s")

    @functools.partial(
        pl.kernel, mesh=mesh,
        out_type=jax.ShapeDtypeStruct((nb, L), jnp.float32),
        compiler_params=pltpu.CompilerParams(use_tc_tiling_on_sc=False),
        scratch_types=[
            pltpu.VMEM((b_per_w,), jnp.int32),
            pltpu.VMEM((b_per_w, L), jnp.float32),
            pltpu.SemaphoreType.DMA,
        ],
    )
    def k(tab_hbm, idx_hbm, out_hbm, idx_v, rows_v, sem):
        wid = lax.axis_index("s") * info.num_cores + lax.axis_index("c")
        base = wid * b_per_w
        pltpu.sync_copy(idx_hbm.at[pl.ds(base, b_per_w)], idx_v)
        copies = []
        for c in range(nchunk):
            copies.append(pltpu.async_copy(
                tab_hbm.at[idx_v.at[pl.ds(c * chunk, chunk)]],
                rows_v.at[pl.ds(c * chunk, chunk)], sem))
        for cp in copies:
            cp.wait()
        pltpu.sync_copy(rows_v, out_hbm.at[pl.ds(base, b_per_w)])

    return k(v16, rsel_flat)


def _k6_body(d_ref, rows_ref, loff_ref, o_ref):
    Q = d_ref.shape[0]
    d = d_ref[...]
    rows = rows_ref[...]
    loff = loff_ref[...]
    io16 = lax.broadcasted_iota(jnp.int32, (Q, 16), 1)
    cols = []
    for j in range(KNN):
        blk = rows[:, j * 16:(j + 1) * 16]
        onehot = io16 == loff[:, j:j + 1]
        cols.append(jnp.sum(jnp.where(onehot, blk, jnp.float32(0.0)),
                            axis=1, keepdims=True))
    v = jnp.concatenate(cols, axis=1)
    w = 1.0 / (d + jnp.float32(DELTA))
    w = w / jnp.sum(w, axis=1, keepdims=True)
    out_nm = jnp.sum(w * v, axis=1, keepdims=True)
    match = d[:, 0:1] == 0.0
    o_ref[...] = jnp.where(match, v[:, 0:1], out_nm)


def _k6_call(dists, rows, loff):
    Q = dists.shape[0]
    return pl.pallas_call(
        _k6_body,
        out_shape=jax.ShapeDtypeStruct((Q, 1), jnp.float32),
    )(dists, rows, loff)


def kernel(key, keys, values):
    Q, D = key.shape
    nkeys = keys.shape[0]
    nblk = (nkeys + KBLK - 1) // KBLK
    ng = nblk * KBLK // GSZ

    e4, mt = _k1_call(key, keys, nkeys)
    rowt = _k2_call(mt)
    rowidx = rowt.T
    cand = _k3_gather(e4.reshape(-1, GSZ), rowidx.reshape(-1))
    cand3 = cand.reshape(Q, NCAND, GSZ)
    dists, rsel, loff = _k4_call(cand3, rowidx, key, ng)
    rows = _k5_gather(values.reshape(-1, 16), rsel.reshape(-1))
    return _k6_call(dists, rows.reshape(Q, KNN * 16), loff)


# speedup vs baseline: 1.5764x; 1.5764x over previous
import functools

import jax
import jax.numpy as jnp
import numpy as np
from jax import lax
from jax.experimental import pallas as pl
from jax.experimental.pallas import tpu as pltpu
from jax.experimental.pallas import tpu_sc as plsc

KNN = 32
DELTA = 1e-3
GSZ = 64
NCAND = 40
KBLK = 2048
BIGF = np.float32(3.0e38)
PADF = np.float32(1.0e30)
BIGI = np.int32(2147483647)


def _k1_body(nkeys, q_ref, k_ref, e_ref, m_ref):
    i = pl.program_id(0)
    q = q_ref[...]
    k = k_ref[...]
    Q = q.shape[0]
    gpb = KBLK // GSZ
    ones = jnp.ones((8, q.shape[1]), jnp.float32)

    acc = lax.dot_general(q, k, (((1,), (1,)), ((), ())),
                          preferred_element_type=jnp.float32)
    ksq_l = lax.dot_general(ones, k * k, (((1,), (1,)), ((), ())),
                            preferred_element_type=jnp.float32,
                            precision=lax.Precision.HIGHEST)[0:1]
    col = i * KBLK + lax.broadcasted_iota(jnp.int32, (1, KBLK), 1)
    pen_l = jnp.where(col >= nkeys, PADF, jnp.float32(0.0))
    e_ref[...] = (ksq_l + pen_l) - 2.0 * acc

    acc_t = lax.dot_general(k, q, (((1,), (1,)), ((), ())),
                            preferred_element_type=jnp.float32)
    ksq_s = jnp.sum(k * k, axis=1, keepdims=True)
    row = i * KBLK + lax.broadcasted_iota(jnp.int32, (KBLK, 1), 0)
    pen_s = jnp.where(row >= nkeys, PADF, jnp.float32(0.0))
    e_t = (ksq_s + pen_s) - 2.0 * acc_t
    m_ref[...] = jnp.min(e_t.reshape(gpb, GSZ, Q), axis=1)


def _k1_call(q, keys_p, nkeys):
    Q, D = q.shape
    npad = keys_p.shape[0]
    nblk = npad // KBLK
    gpb = KBLK // GSZ
    return pl.pallas_call(
        functools.partial(_k1_body, nkeys),
        grid=(nblk,),
        in_specs=[
            pl.BlockSpec((Q, D), lambda i: (0, 0)),
            pl.BlockSpec((KBLK, D), lambda i: (i, 0)),
        ],
        out_specs=[
            pl.BlockSpec((Q, KBLK), lambda i: (0, i)),
            pl.BlockSpec((gpb, Q), lambda i: (i, 0)),
        ],
        out_shape=[
            jax.ShapeDtypeStruct((Q, npad), jnp.float32),
            jax.ShapeDtypeStruct((npad // GSZ, Q), jnp.float32),
        ],
    )(q, keys_p)


def _k2_body(m_ref, rowt_ref, mv_ref):
    ng, Q = m_ref.shape
    mv_ref[...] = m_ref[...]
    rowt_ref[...] = jnp.zeros((NCAND, Q), jnp.int32)

    def body(j, _):
        mv = mv_ref[...]
        iog = lax.broadcasted_iota(jnp.int32, (ng, Q), 0)
        v = jnp.min(mv, axis=0)
        p = jnp.min(jnp.where(mv == v[None, :], iog, BIGI), axis=0)
        ioj = lax.broadcasted_iota(jnp.int32, (NCAND, Q), 0)
        qio = lax.broadcasted_iota(jnp.int32, (NCAND, Q), 1)
        rowt_ref[...] = jnp.where(ioj == j, qio * ng + p[None, :],
                                  rowt_ref[...])
        mv_ref[...] = jnp.where(iog == p[None, :], BIGF, mv)
        return 0

    lax.fori_loop(0, NCAND, body, 0)


def _k2_call(mt):
    ng, Q = mt.shape
    return pl.pallas_call(
        _k2_body,
        out_shape=jax.ShapeDtypeStruct((NCAND, Q), jnp.int32),
        scratch_shapes=[pltpu.VMEM((ng, Q), jnp.float32)],
    )(mt)


def _k3_gather(d2rows, rowidx_flat):
    nrows_out = rowidx_flat.shape[0]
    info = plsc.get_sparse_core_info()
    nw = info.num_cores * info.num_subcores
    b_per_w = nrows_out // nw
    chunk = 128
    nchunk = b_per_w // chunk
    mesh = plsc.VectorSubcoreMesh(core_axis_name="c", subcore_axis_name="s")

    @functools.partial(
        pl.kernel, mesh=mesh,
        out_type=jax.ShapeDtypeStruct((nrows_out, GSZ), jnp.float32),
        compiler_params=pltpu.CompilerParams(use_tc_tiling_on_sc=False),
        scratch_types=[
            pltpu.VMEM((b_per_w,), jnp.int32),
            pltpu.VMEM((b_per_w, GSZ), jnp.float32),
            pltpu.SemaphoreType.DMA,
        ],
    )
    def k(tab_hbm, idx_hbm, out_hbm, idx_v, rows_v, sem):
        wid = lax.axis_index("s") * info.num_cores + lax.axis_index("c")
        pltpu.sync_copy(idx_hbm.at[pl.ds(wid * b_per_w, b_per_w)], idx_v)
        copies = []
        for c in range(nchunk):
            copies.append(pltpu.async_copy(
                tab_hbm.at[idx_v.at[pl.ds(c * chunk, chunk)]],
                rows_v.at[pl.ds(c * chunk, chunk)], sem))
        for cp in copies:
            cp.wait()
        pltpu.sync_copy(rows_v, out_hbm.at[pl.ds(wid * b_per_w, b_per_w)])

    return k(d2rows, rowidx_flat)


def _k4_body(ng, cand_ref, rowidx_ref, q_ref, d_ref, r_ref, l_ref,
             c_ref, p_ref):
    Q, W = cand_ref.shape
    c_ref[...] = cand_ref[...]
    d_ref[...] = jnp.zeros((Q, KNN), jnp.float32)
    p_ref[...] = jnp.zeros((Q, KNN), jnp.int32)

    def body(j, _):
        c = c_ref[...]
        io = lax.broadcasted_iota(jnp.int32, (Q, W), 1)
        v = jnp.min(c, axis=1)
        p = jnp.min(jnp.where(c == v[:, None], io, BIGI), axis=1)
        sel = io == p[:, None]
        ioj = lax.broadcasted_iota(jnp.int32, (Q, KNN), 1)
        d_ref[...] = jnp.where(ioj == j, v[:, None], d_ref[...])
        p_ref[...] = jnp.where(ioj == j, p[:, None], p_ref[...])
        c_ref[...] = jnp.where(sel, BIGF, c)
        return 0

    lax.fori_loop(0, KNN, body, 0)

    qv = q_ref[...]
    qsq = jnp.sum(qv * qv, axis=1, keepdims=True)
    g_tab = (rowidx_ref[...]
             - lax.broadcasted_iota(jnp.int32, (Q, NCAND), 0) * ng)
    pacc = p_ref[...]
    j2 = pacc >> 6
    off = pacc & 63
    gsel = jnp.zeros((Q, KNN), jnp.int32)
    for s in range(NCAND):
        gsel = jnp.where(j2 == s, g_tab[:, s:s + 1], gsel)
    gkey = gsel * GSZ + off
    d_ref[...] = d_ref[...] + qsq
    r_ref[...] = gkey >> 4
    l_ref[...] = gkey & 15


def _k4_call(cand, rowidx, q, ng):
    Q = cand.shape[0]
    W = cand.shape[1]
    return pl.pallas_call(
        functools.partial(_k4_body, ng),
        out_shape=[
            jax.ShapeDtypeStruct((Q, KNN), jnp.float32),
            jax.ShapeDtypeStruct((Q, KNN), jnp.int32),
            jax.ShapeDtypeStruct((Q, KNN), jnp.int32),
        ],
        scratch_shapes=[
            pltpu.VMEM((Q, W), jnp.float32),
            pltpu.VMEM((Q, KNN), jnp.int32),
        ],
    )(cand, rowidx, q)


def _k5_gather(v16, rsel_flat):
    nb = rsel_flat.shape[0]
    L = v16.shape[1]
    info = plsc.get_sparse_core_info()
    nw = info.num_cores * info.num_subcores
    b_per_w = nb // nw
    chunk = 128
    nchunk = b_per_w // chunk
    mesh = plsc.VectorSubcoreMesh(core_axis_name="c", subcore_axis_name="s")

    @functools.partial(
        pl.kernel, mesh=mesh,
        out_type=jax.ShapeDtypeStruct((nb, L), jnp.float32),
        compiler_params=pltpu.CompilerParams(use_tc_tiling_on_sc=False),
        scratch_types=[
            pltpu.VMEM((b_per_w,), jnp.int32),
            pltpu.VMEM((b_per_w, L), jnp.float32),
            pltpu.SemaphoreType.DMA,
        ],
    )
    def k(tab_hbm, idx_hbm, out_hbm, idx_v, rows_v, sem):
        wid = lax.axis_index("s") * info.num_cores + lax.axis_index("c")
        base = wid * b_per_w
        pltpu.sync_copy(idx_hbm.at[pl.ds(base, b_per_w)], idx_v)
        copies = []
        for c in range(nchunk):
            copies.append(pltpu.async_copy(
                tab_hbm.at[idx_v.at[pl.ds(c * chunk, chunk)]],
                rows_v.at[pl.ds(c * chunk, chunk)], sem))
        for cp in copies:
            cp.wait()
        pltpu.sync_copy(rows_v, out_hbm.at[pl.ds(base, b_per_w)])

    return k(v16, rsel_flat)


def _k6_body(d_ref, rows_ref, loff_ref, o_ref):
    Q = d_ref.shape[0]
    d = d_ref[...]
    rows = rows_ref[...]
    loff = loff_ref[...]
    io16 = lax.broadcasted_iota(jnp.int32, (Q, 16), 1)
    cols = []
    for j in range(KNN):
        blk = rows[:, j * 16:(j + 1) * 16]
        onehot = io16 == loff[:, j:j + 1]
        cols.append(jnp.sum(jnp.where(onehot, blk, jnp.float32(0.0)),
                            axis=1, keepdims=True))
    v = jnp.concatenate(cols, axis=1)
    w = 1.0 / (d + jnp.float32(DELTA))
    w = w / jnp.sum(w, axis=1, keepdims=True)
    out_nm = jnp.sum(w * v, axis=1, keepdims=True)
    match = d[:, 0:1] == 0.0
    o_ref[...] = jnp.where(match, v[:, 0:1], out_nm)


def _k6_call(dists, rows, loff):
    Q = dists.shape[0]
    return pl.pallas_call(
        _k6_body,
        out_shape=jax.ShapeDtypeStruct((Q, 1), jnp.float32),
    )(dists, rows, loff)


def kernel(key, keys, values):
    Q, D = key.shape
    nkeys = keys.shape[0]
    npad = ((nkeys + KBLK - 1) // KBLK) * KBLK
    keys_p = jnp.pad(keys, ((0, npad - nkeys), (0, 0)))
    ng = npad // GSZ

    e, mt = _k1_call(key, keys_p, nkeys)
    rowt = _k2_call(mt)
    rowidx = rowt.T
    cand = _k3_gather(e.reshape(Q * ng, GSZ), rowidx.reshape(-1))
    dists, rsel, loff = _k4_call(cand.reshape(Q, NCAND * GSZ), rowidx,
                                 key, ng)
    rows = _k5_gather(values.reshape(-1, 16), rsel.reshape(-1))
    return _k6_call(dists, rows.reshape(Q, KNN * 16), loff)
